# baseline (speedup 1.0000x reference)
import functools

import jax
import jax.numpy as jnp
from jax.experimental import pallas as pl

DENSITY = 0.05


def _mask_count(mask, rows, n):
    m = mask.astype(jnp.int16).reshape(rows, n // 256, 256)
    while m.shape[1] > 1:
        half = m.shape[1] // 2
        m = m[:, :half, :] + m[:, half:, :]
    return jnp.sum(m[:, 0, :].astype(jnp.int32), axis=1, keepdims=True)


def _count_ge(keys16, cand_s16, rows, n):
    return _mask_count(keys16 >= cand_s16, rows, n)


def _kwinner_block(x_ref, o_ref, *, k):
    imin32 = jnp.int32(-2147483648)
    x = x_ref[...]
    rows, n = x.shape
    i = jax.lax.bitcast_convert_type(x, jnp.int32)
    v = jnp.where(i >= 0, i, jnp.bitwise_xor(jnp.bitwise_not(i), imin32))

    hi = jax.lax.shift_right_arithmetic(v, 16).astype(jnp.int16)
    lo = jnp.bitwise_xor(v.astype(jnp.int16), jnp.int16(-32768))

    kk = jnp.int32(k)

    def to_s16(cand_u):
        return jnp.bitwise_xor(cand_u, jnp.int32(0x8000)).astype(jnp.int16)

    def body1(j, t_u):
        bit = jnp.left_shift(jnp.int32(1), 15 - j)
        cand_u = jnp.bitwise_or(t_u, bit)
        cnt = _count_ge(hi, to_s16(cand_u), rows, n)
        return jnp.where(cnt >= kk, cand_u, t_u)

    t_hi_u = jax.lax.fori_loop(0, 16, body1, jnp.zeros((rows, 1), jnp.int32))
    t_hi_s = to_s16(t_hi_u)

    tie = hi == t_hi_s
    c_gt = _mask_count(hi > t_hi_s, rows, n)
    k2 = kk - c_gt

    mlo = jnp.where(tie, lo, jnp.int16(-32768))

    def body2(j, t_u):
        bit = jnp.left_shift(jnp.int32(1), 15 - j)
        cand_u = jnp.bitwise_or(t_u, bit)
        cnt = _count_ge(mlo, to_s16(cand_u), rows, n)
        return jnp.where(cnt >= k2, cand_u, t_u)

    t_lo_u = jax.lax.fori_loop(0, 16, body2, jnp.zeros((rows, 1), jnp.int32))
    t_lo_s = to_s16(t_lo_u)

    keep = jnp.logical_or(hi > t_hi_s, jnp.logical_and(tie, lo >= t_lo_s))
    o_ref[...] = jnp.where(keep, x, 0.0)


@jax.jit
def kernel(x):
    b, n = x.shape
    k = int(n * DENSITY)
    rows_per_block = 8
    grid = (b // rows_per_block,)
    return pl.pallas_call(
        functools.partial(_kwinner_block, k=k),
        grid=grid,
        in_specs=[pl.BlockSpec((rows_per_block, n), lambda i: (i, 0))],
        out_specs=pl.BlockSpec((rows_per_block, n), lambda i: (i, 0)),
        out_shape=jax.ShapeDtypeStruct((b, n), x.dtype),
    )(x)


# speedup vs baseline: 1.4431x; 1.4431x over previous
import functools

import jax
import jax.numpy as jnp
from jax.experimental import pallas as pl

DENSITY = 0.05


def _count_cmp(keys16, cand_s16, rows, n, strict):
    acc = jnp.zeros((rows, 256), jnp.int16)
    for c in range(n // 256):
        blk = keys16[:, c * 256:(c + 1) * 256]
        m = (blk > cand_s16) if strict else (blk >= cand_s16)
        acc = acc + m.astype(jnp.int16)
    return jnp.sum(acc.astype(jnp.int32), axis=1, keepdims=True)


def _count_ge(keys16, cand_s16, rows, n):
    return _count_cmp(keys16, cand_s16, rows, n, strict=False)


def _kwinner_block(x_ref, o_ref, *, k):
    imin32 = jnp.int32(-2147483648)
    x = x_ref[...]
    rows, n = x.shape
    i = jax.lax.bitcast_convert_type(x, jnp.int32)
    v = jnp.where(i >= 0, i, jnp.bitwise_xor(jnp.bitwise_not(i), imin32))

    hi = jax.lax.shift_right_arithmetic(v, 16).astype(jnp.int16)
    lo = jnp.bitwise_xor(v.astype(jnp.int16), jnp.int16(-32768))

    kk = jnp.int32(k)

    def to_s16(cand_u):
        return jnp.bitwise_xor(cand_u, jnp.int32(0x8000)).astype(jnp.int16)

    def body1(j, t_u):
        bit = jnp.left_shift(jnp.int32(1), 15 - j)
        cand_u = jnp.bitwise_or(t_u, bit)
        cnt = _count_ge(hi, to_s16(cand_u), rows, n)
        return jnp.where(cnt >= kk, cand_u, t_u)

    t_hi_u = jax.lax.fori_loop(0, 16, body1, jnp.zeros((rows, 1), jnp.int32))
    t_hi_s = to_s16(t_hi_u)

    tie = hi == t_hi_s
    c_gt = _count_cmp(hi, t_hi_s, rows, n, strict=True)
    k2 = kk - c_gt

    mlo = jnp.where(tie, lo, jnp.int16(-32768))

    def body2(j, t_u):
        bit = jnp.left_shift(jnp.int32(1), 15 - j)
        cand_u = jnp.bitwise_or(t_u, bit)
        cnt = _count_ge(mlo, to_s16(cand_u), rows, n)
        return jnp.where(cnt >= k2, cand_u, t_u)

    t_lo_u = jax.lax.fori_loop(0, 16, body2, jnp.zeros((rows, 1), jnp.int32))
    t_lo_s = to_s16(t_lo_u)

    keep = jnp.logical_or(hi > t_hi_s, jnp.logical_and(tie, lo >= t_lo_s))
    o_ref[...] = jnp.where(keep, x, 0.0)


@jax.jit
def kernel(x):
    b, n = x.shape
    k = int(n * DENSITY)
    rows_per_block = 8
    grid = (b // rows_per_block,)
    return pl.pallas_call(
        functools.partial(_kwinner_block, k=k),
        grid=grid,
        in_specs=[pl.BlockSpec((rows_per_block, n), lambda i: (i, 0))],
        out_specs=pl.BlockSpec((rows_per_block, n), lambda i: (i, 0)),
        out_shape=jax.ShapeDtypeStruct((b, n), x.dtype),
    )(x)


# speedup vs baseline: 1.4453x; 1.0015x over previous
import functools

import jax
import jax.numpy as jnp
from jax.experimental import pallas as pl
from jax.experimental.pallas import tpu as pltpu

DENSITY = 0.05


def _count_cmp(keys16, cand_s16, rows, n, strict):
    acc = jnp.zeros((rows, 256), jnp.int16)
    for c in range(n // 256):
        blk = keys16[:, c * 256:(c + 1) * 256]
        m = (blk > cand_s16) if strict else (blk >= cand_s16)
        acc = acc + m.astype(jnp.int16)
    return jnp.sum(acc.astype(jnp.int32), axis=1, keepdims=True)


def _count_ge(keys16, cand_s16, rows, n):
    return _count_cmp(keys16, cand_s16, rows, n, strict=False)


def _kwinner_block(x_ref, o_ref, *, k):
    imin32 = jnp.int32(-2147483648)
    x = x_ref[...]
    rows, n = x.shape
    i = jax.lax.bitcast_convert_type(x, jnp.int32)
    v = jnp.where(i >= 0, i, jnp.bitwise_xor(jnp.bitwise_not(i), imin32))

    hi = jax.lax.shift_right_arithmetic(v, 16).astype(jnp.int16)
    lo = jnp.bitwise_xor(v.astype(jnp.int16), jnp.int16(-32768))

    kk = jnp.int32(k)

    def to_s16(cand_u):
        return jnp.bitwise_xor(cand_u, jnp.int32(0x8000)).astype(jnp.int16)

    def body1(j, t_u):
        bit = jnp.left_shift(jnp.int32(1), 15 - j)
        cand_u = jnp.bitwise_or(t_u, bit)
        cnt = _count_ge(hi, to_s16(cand_u), rows, n)
        return jnp.where(cnt >= kk, cand_u, t_u)

    t_hi_u = jax.lax.fori_loop(0, 16, body1, jnp.zeros((rows, 1), jnp.int32))
    t_hi_s = to_s16(t_hi_u)

    tie = hi == t_hi_s
    c_gt = _count_cmp(hi, t_hi_s, rows, n, strict=True)
    k2 = kk - c_gt

    mlo = jnp.where(tie, lo, jnp.int16(-32768))

    def body2(j, t_u):
        bit = jnp.left_shift(jnp.int32(1), 15 - j)
        cand_u = jnp.bitwise_or(t_u, bit)
        cnt = _count_ge(mlo, to_s16(cand_u), rows, n)
        return jnp.where(cnt >= k2, cand_u, t_u)

    t_lo_u = jax.lax.fori_loop(0, 16, body2, jnp.zeros((rows, 1), jnp.int32))
    t_lo_s = to_s16(t_lo_u)

    keep = jnp.logical_or(hi > t_hi_s, jnp.logical_and(tie, lo >= t_lo_s))
    o_ref[...] = jnp.where(keep, x, 0.0)


@jax.jit
def kernel(x):
    b, n = x.shape
    k = int(n * DENSITY)
    rows_per_block = 8
    grid = (b // rows_per_block,)
    return pl.pallas_call(
        functools.partial(_kwinner_block, k=k),
        grid=grid,
        in_specs=[pl.BlockSpec((rows_per_block, n), lambda i: (i, 0))],
        out_specs=pl.BlockSpec((rows_per_block, n), lambda i: (i, 0)),
        out_shape=jax.ShapeDtypeStruct((b, n), x.dtype),
        compiler_params=pltpu.CompilerParams(
            dimension_semantics=("parallel",)),
    )(x)


# speedup vs baseline: 1.6490x; 1.1409x over previous
import functools

import jax
import jax.numpy as jnp
from jax.experimental import pallas as pl
from jax.experimental.pallas import tpu as pltpu

DENSITY = 0.05


def _count_cmp(keys16, cand_s16, rows, n, strict):
    n_acc = 8
    accs = [jnp.zeros((rows, 256), jnp.int16) for _ in range(n_acc)]
    for c in range(n // 256):
        blk = keys16[:, c * 256:(c + 1) * 256]
        m = (blk > cand_s16) if strict else (blk >= cand_s16)
        accs[c % n_acc] = accs[c % n_acc] + m.astype(jnp.int16)
    while len(accs) > 1:
        accs = [a + b for a, b in zip(accs[::2], accs[1::2])]
    return jnp.sum(accs[0].astype(jnp.int32), axis=1, keepdims=True)


def _count_ge(keys16, cand_s16, rows, n):
    return _count_cmp(keys16, cand_s16, rows, n, strict=False)


def _kwinner_block(x_ref, o_ref, *, k):
    imin32 = jnp.int32(-2147483648)
    x = x_ref[...]
    rows, n = x.shape
    i = jax.lax.bitcast_convert_type(x, jnp.int32)
    v = jnp.where(i >= 0, i, jnp.bitwise_xor(jnp.bitwise_not(i), imin32))

    hi = jax.lax.shift_right_arithmetic(v, 16).astype(jnp.int16)
    lo = jnp.bitwise_xor(v.astype(jnp.int16), jnp.int16(-32768))

    kk = jnp.int32(k)

    def to_s16(cand_u):
        return jnp.bitwise_xor(cand_u, jnp.int32(0x8000)).astype(jnp.int16)

    def body1(j, t_u):
        bit = jnp.left_shift(jnp.int32(1), 15 - j)
        cand_u = jnp.bitwise_or(t_u, bit)
        cnt = _count_ge(hi, to_s16(cand_u), rows, n)
        return jnp.where(cnt >= kk, cand_u, t_u)

    t_hi_u = jax.lax.fori_loop(0, 16, body1, jnp.zeros((rows, 1), jnp.int32))
    t_hi_s = to_s16(t_hi_u)

    tie = hi == t_hi_s
    c_gt = _count_cmp(hi, t_hi_s, rows, n, strict=True)
    k2 = kk - c_gt

    mlo = jnp.where(tie, lo, jnp.int16(-32768))

    def body2(j, t_u):
        bit = jnp.left_shift(jnp.int32(1), 15 - j)
        cand_u = jnp.bitwise_or(t_u, bit)
        cnt = _count_ge(mlo, to_s16(cand_u), rows, n)
        return jnp.where(cnt >= k2, cand_u, t_u)

    t_lo_u = jax.lax.fori_loop(0, 16, body2, jnp.zeros((rows, 1), jnp.int32))
    t_lo_s = to_s16(t_lo_u)

    keep = jnp.logical_or(hi > t_hi_s, jnp.logical_and(tie, lo >= t_lo_s))
    o_ref[...] = jnp.where(keep, x, 0.0)


@jax.jit
def kernel(x):
    b, n = x.shape
    k = int(n * DENSITY)
    rows_per_block = 8
    grid = (b // rows_per_block,)
    return pl.pallas_call(
        functools.partial(_kwinner_block, k=k),
        grid=grid,
        in_specs=[pl.BlockSpec((rows_per_block, n), lambda i: (i, 0))],
        out_specs=pl.BlockSpec((rows_per_block, n), lambda i: (i, 0)),
        out_shape=jax.ShapeDtypeStruct((b, n), x.dtype),
        compiler_params=pltpu.CompilerParams(
            dimension_semantics=("parallel",)),
    )(x)
